# baseline (speedup 1.0000x reference)
import functools

import jax
import jax.numpy as jnp
from jax import lax
from jax.experimental import pallas as pl
from jax.experimental.pallas import tpu as pltpu
from jax.experimental.pallas import tpu_sc as plsc

N = 10000
E = 320000
D = 128
EPS = 1e-5

NC = 2
NS = 16
HD = D // NC
ET = E // NS
CH = 80
NCH = ET // CH
N_PAD = 10112
STRIPE = N_PAD // NS
XROWS = N // NS


def _sc_segment_sum(x2, src2, dst2):
    mesh = plsc.VectorSubcoreMesh(core_axis_name="c", subcore_axis_name="s")

    @functools.partial(
        pl.kernel,
        mesh=mesh,
        out_type=jax.ShapeDtypeStruct((NC, N_PAD, HD), jnp.float32),
        compiler_params=pltpu.CompilerParams(use_tc_tiling_on_sc=False),
        scratch_types=[
            pltpu.VMEM((ET,), jnp.int32),
            pltpu.VMEM((ET,), jnp.int32),
            pltpu.VMEM((CH, HD), jnp.float32),
            pltpu.VMEM((CH, HD), jnp.float32),
            pltpu.VMEM_SHARED((N, HD), jnp.float32),
            pltpu.VMEM_SHARED((N_PAD, HD), jnp.float32),
            pltpu.SemaphoreType.DMA,
            pltpu.SemaphoreType.DMA,
            pltpu.SemaphoreType.DMA,
            pltpu.SemaphoreType.DMA,
        ],
    )
    def k(x_hbm, src_hbm, dst_hbm, out_hbm,
          src_v, dst_v, rows0_v, rows1_v, x_sh, agg_sh,
          sem0, sem1, semz, semx):
        c = lax.axis_index("c")
        s = lax.axis_index("s")
        pltpu.async_copy(src_hbm.at[s], src_v, sem0)
        pltpu.async_copy(dst_hbm.at[s], dst_v, sem1)
        pltpu.async_copy(x_hbm.at[c, pl.ds(s * XROWS, XROWS)],
                         x_sh.at[pl.ds(s * XROWS, XROWS)], semx)

        zero16 = jnp.zeros((16,), jnp.float32)

        def zbody(i, carry):
            for kcol in range(HD // 16):
                rows0_v[i, pl.ds(kcol * 16, 16)] = zero16
            return carry

        lax.fori_loop(0, CH, zbody, 0)
        nfull_z = STRIPE // CH
        rem_z = STRIPE - nfull_z * CH
        for kz in range(nfull_z):
            pltpu.async_copy(
                rows0_v, agg_sh.at[pl.ds(s * STRIPE + kz * CH, CH)], semz)
        pltpu.async_copy(
            rows0_v.at[pl.ds(0, rem_z)],
            agg_sh.at[pl.ds(s * STRIPE + nfull_z * CH, rem_z)], semz)
        for kz in range(nfull_z):
            pltpu.make_async_copy(
                rows0_v, agg_sh.at[pl.ds(s * STRIPE + kz * CH, CH)],
                semz).wait()
        pltpu.make_async_copy(
            rows0_v.at[pl.ds(0, rem_z)],
            agg_sh.at[pl.ds(s * STRIPE + nfull_z * CH, rem_z)], semz).wait()
        pltpu.make_async_copy(src_hbm.at[s], src_v, sem0).wait()
        pltpu.make_async_copy(dst_hbm.at[s], dst_v, sem1).wait()
        pltpu.make_async_copy(x_hbm.at[c, pl.ds(s * XROWS, XROWS)],
                              x_sh.at[pl.ds(s * XROWS, XROWS)], semx).wait()
        plsc.subcore_barrier()

        def start(j, buf, sem):
            pltpu.async_copy(x_sh.at[src_v.at[pl.ds(j * CH, CH)]], buf, sem)

        def finish(j, buf, sem):
            pltpu.make_async_copy(
                x_sh.at[src_v.at[pl.ds(j * CH, CH)]], buf, sem).wait()
            pltpu.sync_copy(buf, agg_sh.at[dst_v.at[pl.ds(j * CH, CH)]],
                            add=True)

        start(0, rows0_v, sem0)
        start(1, rows1_v, sem1)

        def body(jj, carry):
            j0 = 2 * jj
            finish(j0, rows0_v, sem0)
            start(j0 + 2, rows0_v, sem0)
            finish(j0 + 1, rows1_v, sem1)
            start(j0 + 3, rows1_v, sem1)
            return carry

        lax.fori_loop(0, NCH // 2 - 1, body, 0)
        finish(NCH - 2, rows0_v, sem0)
        finish(NCH - 1, rows1_v, sem1)
        plsc.subcore_barrier()
        pltpu.sync_copy(agg_sh.at[pl.ds(s * STRIPE, STRIPE)],
                        out_hbm.at[c, pl.ds(s * STRIPE, STRIPE)])

    return k(x2, src2, dst2)


def _tc_finish(halves, wt, x, bn_w, bn_b):
    def body(p_ref, wt_ref, x_ref, w_ref, b_ref, o_ref):
        agg = jnp.concatenate([p_ref[0, :N, :], p_ref[1, :N, :]], axis=1)
        h = jnp.dot(agg, wt_ref[...], preferred_element_type=jnp.float32)
        mean = jnp.mean(h, axis=0, keepdims=True)
        var = jnp.mean(h * h, axis=0, keepdims=True) - mean * mean
        o = (h - mean) * (lax.rsqrt(var + EPS) * w_ref[...]) + b_ref[...]
        o_ref[...] = jnp.maximum(o, 0.0) + x_ref[...]

    return pl.pallas_call(
        body,
        out_shape=jax.ShapeDtypeStruct((N, D), jnp.float32),
    )(halves, wt, x, bn_w.reshape(1, D), bn_b.reshape(1, D))


def kernel(x, edge_index, W, b, bn_weight, bn_bias):
    del b
    src2 = edge_index[0].reshape(NS, ET)
    dst2 = edge_index[1].reshape(NS, ET)
    x2 = x.reshape(N, NC, HD).transpose(1, 0, 2)
    halves = _sc_segment_sum(x2, src2, dst2)
    return _tc_finish(halves, W.T, x, bn_weight, bn_bias)


# speedup vs baseline: 1.3382x; 1.3382x over previous
import functools

import jax
import jax.numpy as jnp
from jax import lax
from jax.experimental import pallas as pl
from jax.experimental.pallas import tpu as pltpu
from jax.experimental.pallas import tpu_sc as plsc

N = 10000
E = 320000
D = 128
EPS = 1e-5

NC = 2
NS = 16
EW = E // (NC * NS)
CH = 160
IB = 10 * CH
NB = EW // IB
REM = EW - NB * IB
N_PAD = 10112
STRIPE = N_PAD // NS


def _sc_segment_sum(x, src4, dst4):
    mesh = plsc.VectorSubcoreMesh(core_axis_name="c", subcore_axis_name="s")

    @functools.partial(
        pl.kernel,
        mesh=mesh,
        out_type=jax.ShapeDtypeStruct((NC, N_PAD, D), jnp.float32),
        compiler_params=pltpu.CompilerParams(use_tc_tiling_on_sc=False),
        scratch_types=[
            pltpu.VMEM((IB,), jnp.int32),
            pltpu.VMEM((IB,), jnp.int32),
            pltpu.VMEM((IB,), jnp.int32),
            pltpu.VMEM((IB,), jnp.int32),
            pltpu.VMEM((CH, D), jnp.float32),
            pltpu.VMEM((CH, D), jnp.float32),
            pltpu.VMEM_SHARED((N_PAD, D), jnp.float32),
            pltpu.SemaphoreType.DMA,
            pltpu.SemaphoreType.DMA,
            pltpu.SemaphoreType.DMA,
            pltpu.SemaphoreType.DMA,
            pltpu.SemaphoreType.DMA,
        ],
    )
    def k(x_hbm, src_hbm, dst_hbm, out_hbm,
          srcA, dstA, srcB, dstB, rows0_v, rows1_v, agg_sh,
          sem0, sem1, semz, semA, semB):
        c = lax.axis_index("c")
        s = lax.axis_index("s")
        idx_bufs = [(srcA, dstA, semA), (srcB, dstB, semB)]

        def stage(b, size):
            sbuf, dbuf, sem = idx_bufs[b % 2]
            pltpu.async_copy(src_hbm.at[c, s, pl.ds(b * IB, size)],
                             sbuf.at[pl.ds(0, size)], sem)
            pltpu.async_copy(dst_hbm.at[c, s, pl.ds(b * IB, size)],
                             dbuf.at[pl.ds(0, size)], sem)

        def stage_wait(b, size):
            sbuf, dbuf, sem = idx_bufs[b % 2]
            pltpu.make_async_copy(src_hbm.at[c, s, pl.ds(b * IB, size)],
                                  sbuf.at[pl.ds(0, size)], sem).wait()
            pltpu.make_async_copy(dst_hbm.at[c, s, pl.ds(b * IB, size)],
                                  dbuf.at[pl.ds(0, size)], sem).wait()

        stage(0, IB)
        stage(1, IB)

        zero16 = jnp.zeros((16,), jnp.float32)

        def zbody(i, carry):
            for kcol in range(D // 16):
                rows0_v[i, pl.ds(kcol * 16, 16)] = zero16
            return carry

        lax.fori_loop(0, CH, zbody, 0)
        nfull_z = STRIPE // CH
        rem_z = STRIPE - nfull_z * CH
        for kz in range(nfull_z):
            pltpu.async_copy(
                rows0_v, agg_sh.at[pl.ds(s * STRIPE + kz * CH, CH)], semz)
        pltpu.async_copy(
            rows0_v.at[pl.ds(0, rem_z)],
            agg_sh.at[pl.ds(s * STRIPE + nfull_z * CH, rem_z)], semz)
        for kz in range(nfull_z):
            pltpu.make_async_copy(
                rows0_v, agg_sh.at[pl.ds(s * STRIPE + kz * CH, CH)],
                semz).wait()
        pltpu.make_async_copy(
            rows0_v.at[pl.ds(0, rem_z)],
            agg_sh.at[pl.ds(s * STRIPE + nfull_z * CH, rem_z)], semz).wait()
        plsc.subcore_barrier()

        def start(sbuf, j, size, buf, sem):
            pltpu.async_copy(
                x_hbm.at[sbuf.at[pl.ds(j * CH, size)]],
                buf.at[pl.ds(0, size)], sem)

        def finish(sbuf, dbuf, j, size, buf, sem):
            pltpu.make_async_copy(
                x_hbm.at[sbuf.at[pl.ds(j * CH, size)]],
                buf.at[pl.ds(0, size)], sem).wait()
            pltpu.sync_copy(buf.at[pl.ds(0, size)],
                            agg_sh.at[dbuf.at[pl.ds(j * CH, size)]],
                            add=True)

        def process_block(parity, prefetch_b, prefetch_size):
            sbuf, dbuf, _ = idx_bufs[parity]
            if prefetch_b is not None:
                stage(prefetch_b, prefetch_size)
            start(sbuf, 0, CH, rows0_v, sem0)
            start(sbuf, 1, CH, rows1_v, sem1)

            def body(jj, carry):
                j0 = 2 * jj
                finish(sbuf, dbuf, j0, CH, rows0_v, sem0)
                start(sbuf, j0 + 2, CH, rows0_v, sem0)
                finish(sbuf, dbuf, j0 + 1, CH, rows1_v, sem1)
                start(sbuf, j0 + 3, CH, rows1_v, sem1)
                return carry

            lax.fori_loop(0, 4, body, 0)
            finish(sbuf, dbuf, 8, CH, rows0_v, sem0)
            finish(sbuf, dbuf, 9, CH, rows1_v, sem1)

        stage_wait(0, IB)
        process_block(0, None, 0)
        for b in range(1, NB):
            stage_wait(b, IB)
            nxt = b + 1
            process_block(b % 2, nxt, IB if nxt < NB else REM)
        stage_wait(NB, REM)
        sbuf, dbuf, _ = idx_bufs[NB % 2]
        start(sbuf, 0, CH, rows0_v, sem0)
        start(sbuf, 1, CH, rows1_v, sem1)
        finish(sbuf, dbuf, 0, CH, rows0_v, sem0)
        start(sbuf, 2, REM - 2 * CH, rows0_v, sem0)
        finish(sbuf, dbuf, 1, CH, rows1_v, sem1)
        finish(sbuf, dbuf, 2, REM - 2 * CH, rows0_v, sem0)
        plsc.subcore_barrier()
        pltpu.sync_copy(agg_sh.at[pl.ds(s * STRIPE, STRIPE)],
                        out_hbm.at[c, pl.ds(s * STRIPE, STRIPE)])

    return k(x, src4, dst4)


def _tc_finish(partials, wt, x, bn_w, bn_b):
    def body(p_ref, wt_ref, x_ref, w_ref, b_ref, o_ref):
        agg = p_ref[0, :N, :] + p_ref[1, :N, :]
        h = jnp.dot(agg, wt_ref[...], preferred_element_type=jnp.float32)
        mean = jnp.mean(h, axis=0, keepdims=True)
        var = jnp.mean(h * h, axis=0, keepdims=True) - mean * mean
        o = (h - mean) * (lax.rsqrt(var + EPS) * w_ref[...]) + b_ref[...]
        o_ref[...] = jnp.maximum(o, 0.0) + x_ref[...]

    return pl.pallas_call(
        body,
        out_shape=jax.ShapeDtypeStruct((N, D), jnp.float32),
    )(partials, wt, x, bn_w.reshape(1, D), bn_b.reshape(1, D))


def kernel(x, edge_index, W, b, bn_weight, bn_bias):
    del b
    src4 = edge_index[0].reshape(NC, NS, EW)
    dst4 = edge_index[1].reshape(NC, NS, EW)
    partials = _sc_segment_sum(x, src4, dst4)
    return _tc_finish(partials, W.T, x, bn_weight, bn_bias)


# speedup vs baseline: 1.4051x; 1.0499x over previous
import functools

import jax
import jax.numpy as jnp
from jax import lax
from jax.experimental import pallas as pl
from jax.experimental.pallas import tpu as pltpu
from jax.experimental.pallas import tpu_sc as plsc

N = 10000
E = 320000
D = 128
EPS = 1e-5

NC = 2
NS = 16
EW = E // (NC * NS)
CH = 112
NFULL = EW // CH
TAIL = EW - NFULL * CH
N_PAD = 10112
STRIPE = N_PAD // NS


def _sc_segment_sum(x, src4, dst4):
    mesh = plsc.VectorSubcoreMesh(core_axis_name="c", subcore_axis_name="s")

    @functools.partial(
        pl.kernel,
        mesh=mesh,
        out_type=jax.ShapeDtypeStruct((NC, N_PAD, D), jnp.float32),
        compiler_params=pltpu.CompilerParams(use_tc_tiling_on_sc=False),
        scratch_types=[
            pltpu.VMEM((EW,), jnp.int32),
            pltpu.VMEM((EW,), jnp.int32),
            pltpu.VMEM((CH, D), jnp.float32),
            pltpu.VMEM((CH, D), jnp.float32),
            pltpu.VMEM_SHARED((N_PAD, D), jnp.float32),
            pltpu.SemaphoreType.DMA,
            pltpu.SemaphoreType.DMA,
            pltpu.SemaphoreType.DMA,
        ],
    )
    def k(x_hbm, src_hbm, dst_hbm, out_hbm,
          src_v, dst_v, rows0_v, rows1_v, agg_sh, sem0, sem1, semz):
        c = lax.axis_index("c")
        s = lax.axis_index("s")
        pltpu.async_copy(src_hbm.at[c, s], src_v, sem0)
        pltpu.async_copy(dst_hbm.at[c, s], dst_v, sem1)

        zero16 = jnp.zeros((16,), jnp.float32)

        def zbody(i, carry):
            for kcol in range(D // 16):
                rows0_v[i, pl.ds(kcol * 16, 16)] = zero16
            return carry

        lax.fori_loop(0, CH, zbody, 0)
        nfull_z = STRIPE // CH
        rem_z = STRIPE - nfull_z * CH
        for kz in range(nfull_z):
            pltpu.async_copy(
                rows0_v, agg_sh.at[pl.ds(s * STRIPE + kz * CH, CH)], semz)
        pltpu.async_copy(
            rows0_v.at[pl.ds(0, rem_z)],
            agg_sh.at[pl.ds(s * STRIPE + nfull_z * CH, rem_z)], semz)
        for kz in range(nfull_z):
            pltpu.make_async_copy(
                rows0_v, agg_sh.at[pl.ds(s * STRIPE + kz * CH, CH)],
                semz).wait()
        pltpu.make_async_copy(
            rows0_v.at[pl.ds(0, rem_z)],
            agg_sh.at[pl.ds(s * STRIPE + nfull_z * CH, rem_z)], semz).wait()
        pltpu.make_async_copy(src_hbm.at[c, s], src_v, sem0).wait()
        pltpu.make_async_copy(dst_hbm.at[c, s], dst_v, sem1).wait()
        plsc.subcore_barrier()

        def start(j, buf, sem):
            pltpu.async_copy(x_hbm.at[src_v.at[pl.ds(j * CH, CH)]], buf, sem)

        def finish(j, buf, sem):
            pltpu.make_async_copy(
                x_hbm.at[src_v.at[pl.ds(j * CH, CH)]], buf, sem).wait()
            pltpu.sync_copy(buf, agg_sh.at[dst_v.at[pl.ds(j * CH, CH)]],
                            add=True)

        start(0, rows0_v, sem0)
        start(1, rows1_v, sem1)

        def body(jj, carry):
            j0 = 2 * jj
            finish(j0, rows0_v, sem0)
            start(j0 + 2, rows0_v, sem0)
            finish(j0 + 1, rows1_v, sem1)
            start(j0 + 3, rows1_v, sem1)
            return carry

        lax.fori_loop(0, (NFULL - 3) // 2, body, 0)
        finish(NFULL - 3, rows0_v, sem0)
        start(NFULL - 1, rows0_v, sem0)
        finish(NFULL - 2, rows1_v, sem1)
        tail0 = NFULL * CH
        tbuf = rows1_v.at[pl.ds(0, TAIL)]
        pltpu.async_copy(x_hbm.at[src_v.at[pl.ds(tail0, TAIL)]], tbuf, sem1)
        finish(NFULL - 1, rows0_v, sem0)
        pltpu.make_async_copy(
            x_hbm.at[src_v.at[pl.ds(tail0, TAIL)]], tbuf, sem1).wait()
        pltpu.sync_copy(tbuf, agg_sh.at[dst_v.at[pl.ds(tail0, TAIL)]],
                        add=True)
        plsc.subcore_barrier()
        pltpu.sync_copy(agg_sh.at[pl.ds(s * STRIPE, STRIPE)],
                        out_hbm.at[c, pl.ds(s * STRIPE, STRIPE)])

    return k(x, src4, dst4)


def _tc_finish(partials, wt, x, bn_w, bn_b):
    def body(p_ref, wt_ref, x_ref, w_ref, b_ref, o_ref):
        agg = p_ref[0, :N, :] + p_ref[1, :N, :]
        h = jnp.dot(agg, wt_ref[...], preferred_element_type=jnp.float32)
        mean = jnp.mean(h, axis=0, keepdims=True)
        var = jnp.mean(h * h, axis=0, keepdims=True) - mean * mean
        o = (h - mean) * (lax.rsqrt(var + EPS) * w_ref[...]) + b_ref[...]
        o_ref[...] = jnp.maximum(o, 0.0) + x_ref[...]

    return pl.pallas_call(
        body,
        out_shape=jax.ShapeDtypeStruct((N, D), jnp.float32),
    )(partials, wt, x, bn_w.reshape(1, D), bn_b.reshape(1, D))


def kernel(x, edge_index, W, b, bn_weight, bn_bias):
    del b
    src4 = edge_index[0].reshape(NC, NS, EW)
    dst4 = edge_index[1].reshape(NC, NS, EW)
    partials = _sc_segment_sum(x, src4, dst4)
    return _tc_finish(partials, W.T, x, bn_weight, bn_bias)
